# baseline (speedup 1.0000x reference)
import functools

import jax
import jax.numpy as jnp
from jax import lax
from jax.experimental import pallas as pl
from jax.experimental.pallas import tpu as pltpu
from jax.experimental.pallas import tpu_sc as plsc

_D = 1024
_H = 4096
_E = 8
_N = 2048
_NP = 2 * _N
_BLK = 256
_NBLK = 24
_NSLOT = _NBLK * _BLK
_NH = 4
_HT = _H // _NH


def _router_body(xf_ref, wr_ref, br_ref, pos_ref, grep_ref, be_ref, aux_ref):
    xf = xf_ref[...]
    logits = jnp.dot(xf, wr_ref[...], preferred_element_type=jnp.float32)
    logits = logits + br_ref[...]
    m = jnp.max(logits, axis=1, keepdims=True)
    ex = jnp.exp(logits - m)
    probs = ex / jnp.sum(ex, axis=1, keepdims=True)

    lane = lax.broadcasted_iota(jnp.int32, (_N, _E), 1)
    i1 = jnp.argmax(probs, axis=1, keepdims=True).astype(jnp.int32)
    v1 = jnp.max(probs, axis=1, keepdims=True)
    masked = jnp.where(lane == i1, -1.0, probs)
    i2 = jnp.argmax(masked, axis=1, keepdims=True).astype(jnp.int32)
    v2 = jnp.max(masked, axis=1, keepdims=True)
    tot = v1 + v2
    g0 = v1 / tot
    g1 = v2 / tot

    one1 = (lane == i1).astype(jnp.float32)
    f = jnp.sum(one1, axis=0, keepdims=True) * (1.0 / _N)
    pbar = jnp.sum(probs, axis=0, keepdims=True) * (1.0 / _N)
    aux_ref[...] = jnp.reshape(float(_E) * jnp.sum(f * pbar), (1, 1))

    e_pair = jnp.concatenate([i1, i2], axis=0)
    g_pair = jnp.concatenate([g0, g1], axis=0)
    lane_p = lax.broadcasted_iota(jnp.int32, (_NP, _E), 1)
    oh = (lane_p == e_pair).astype(jnp.float32)

    counts = jnp.sum(oh, axis=0, keepdims=True)
    padded = jnp.ceil(counts * (1.0 / _BLK)) * float(_BLK)
    ia = lax.broadcasted_iota(jnp.int32, (_E, _E), 0)
    ja = lax.broadcasted_iota(jnp.int32, (_E, _E), 1)
    tri_e = (ia < ja).astype(jnp.float32)
    base = jnp.dot(padded, tri_e, preferred_element_type=jnp.float32)

    ch = 512
    ic = lax.broadcasted_iota(jnp.int32, (ch, ch), 0)
    jc = lax.broadcasted_iota(jnp.int32, (ch, ch), 1)
    tri_c = (jc < ic).astype(jnp.float32)
    carry = jnp.zeros((1, _E), jnp.float32)
    ranks = []
    for c in range(_NP // ch):
        blk = lax.slice(oh, (c * ch, 0), ((c + 1) * ch, _E))
        ranks.append(jnp.dot(tri_c, blk, preferred_element_type=jnp.float32)
                     + carry)
        carry = carry + jnp.sum(blk, axis=0, keepdims=True)
    rank = jnp.concatenate(ranks, axis=0)

    posf = jnp.sum((rank + base) * oh, axis=1, keepdims=True)
    pos_ref[...] = posf.astype(jnp.int32)
    grep_ref[...] = g_pair * jnp.ones((1, 128), jnp.float32)

    bv = (lax.broadcasted_iota(jnp.int32, (32, _E), 0) * _BLK).astype(
        jnp.float32)
    be = jnp.sum((base <= bv).astype(jnp.int32), axis=1, keepdims=True) - 1
    lane_e = lax.broadcasted_iota(jnp.int32, (32, _E), 1)
    has = (jnp.ones((32, 1), jnp.float32) * counts) > 0.0
    elast = jnp.max(jnp.where(has, lane_e, 0), axis=1, keepdims=True)
    nreal = jnp.sum(padded, axis=1, keepdims=True) * (1.0 / _BLK)
    row = lax.broadcasted_iota(jnp.int32, (32, 1), 0)
    meta = jnp.where(row == 24, nreal.astype(jnp.int32)[0:1, 0:1],
                     jnp.minimum(be, elast))
    be_ref[...] = meta


_router = pl.pallas_call(
    _router_body,
    out_shape=(
        jax.ShapeDtypeStruct((_NP, 1), jnp.int32),
        jax.ShapeDtypeStruct((_NP, 128), jnp.float32),
        jax.ShapeDtypeStruct((32, 1), jnp.int32),
        jax.ShapeDtypeStruct((1, 1), jnp.float32),
    ),
)


@functools.cache
def _make_dispatch():
    mesh = plsc.VectorSubcoreMesh(core_axis_name="c", subcore_axis_name="s")

    @functools.partial(
        pl.kernel,
        out_type=(
            jax.ShapeDtypeStruct((_NSLOT, _D), jnp.float32),
            jax.ShapeDtypeStruct((_NSLOT, 128), jnp.float32),
        ),
        mesh=mesh,
        scratch_types=[
            pltpu.VMEM((32,), jnp.int32),
            pltpu.VMEM((32,), jnp.int32),
            pltpu.VMEM((32, _D), jnp.float32),
            pltpu.VMEM((32, _D), jnp.float32),
            pltpu.VMEM((128,), jnp.int32),
            pltpu.VMEM((128, 128), jnp.float32),
            pltpu.SemaphoreType.DMA,
            pltpu.SemaphoreType.DMA,
            pltpu.SemaphoreType.DMA,
        ],
    )
    def _dispatch(xf_hbm, pos_hbm, grep_hbm, xs_hbm, gs_hbm,
                  idx0_v, idx1_v, rows0_v, rows1_v, idxg_v, grows_v,
                  sem0, sem1, semg):
        cid = lax.axis_index("c")
        sid = lax.axis_index("s")
        wid = sid * 2 + cid
        src0 = (wid % 16) * 128

        pltpu.sync_copy(pos_hbm.at[wid], idxg_v)
        pltpu.sync_copy(grep_hbm.at[wid], grows_v)
        gcp = pltpu.async_copy(grows_v, gs_hbm.at[idxg_v], semg)

        idxs = (idx0_v, idx1_v)
        rows = (rows0_v, rows1_v)
        sems = (sem0, sem1)
        cps = [None, None]
        for c in range(4):
            bi = c % 2
            if cps[bi] is not None:
                cps[bi].wait()
            pltpu.sync_copy(pos_hbm.at[wid, pl.ds(c * 32, 32)], idxs[bi])
            pltpu.sync_copy(xf_hbm.at[pl.ds(src0 + c * 32, 32)], rows[bi])
            cps[bi] = pltpu.async_copy(rows[bi], xs_hbm.at[idxs[bi]], sems[bi])
        gcp.wait()
        cps[0].wait()
        cps[1].wait()

    return _dispatch


_NHA = 2
_HTA = _H // _NHA


def _ffn_a_body(be_ref, xs_ref, w1_ref, b1_ref, w2_ref, b2_ref, g_ref):
    b = pl.program_id(1)

    @pl.when(b < be_ref[24])
    def _():
        xv = xs_ref[...]
        h1 = jnp.dot(xv, w1_ref[0], preferred_element_type=jnp.float32)
        h1 = h1 + b1_ref[0]
        h2 = jnp.dot(xv, w2_ref[0], preferred_element_type=jnp.float32)
        h2 = h2 + b2_ref[0]
        g_ref[...] = (h1 * jax.nn.sigmoid(h1) * h2).astype(jnp.bfloat16)


_ffn_a = pl.pallas_call(
    _ffn_a_body,
    grid_spec=pltpu.PrefetchScalarGridSpec(
        num_scalar_prefetch=1,
        grid=(_NHA, _NBLK),
        in_specs=[
            pl.BlockSpec((_BLK, _D),
                         lambda h, b, be: (jnp.minimum(b, be[24] - 1), 0)),
            pl.BlockSpec((1, _D, _HTA), lambda h, b, be: (be[b], 0, h)),
            pl.BlockSpec((1, 1, _HTA), lambda h, b, be: (be[b], 0, h)),
            pl.BlockSpec((1, _D, _HTA), lambda h, b, be: (be[b], 0, h)),
            pl.BlockSpec((1, 1, _HTA), lambda h, b, be: (be[b], 0, h)),
        ],
        out_specs=pl.BlockSpec(
            (_BLK, _HTA),
            lambda h, b, be: (jnp.where(b < be[24], b, _NBLK - 1), h)),
    ),
    out_shape=jax.ShapeDtypeStruct((_NSLOT, _H), jnp.bfloat16),
    compiler_params=pltpu.CompilerParams(
        dimension_semantics=("arbitrary", "arbitrary")),
)


def _ffn_b_body(be_ref, g_ref, w3_ref, b3_ref, gs_ref, ys_ref, acc_ref):
    b = pl.program_id(0)
    k = pl.program_id(1)

    @pl.when(b < be_ref[24])
    def _():
        part = jnp.dot(g_ref[...], w3_ref[0],
                       preferred_element_type=jnp.float32)

        @pl.when(k == 0)
        def _():
            acc_ref[...] = part

        @pl.when(k > 0)
        def _():
            acc_ref[...] = acc_ref[...] + part

        @pl.when(k == 3)
        def _():
            ys_ref[...] = (acc_ref[...] + b3_ref[0]) * gs_ref[:, 0:1]


_ffn_b = pl.pallas_call(
    _ffn_b_body,
    grid_spec=pltpu.PrefetchScalarGridSpec(
        num_scalar_prefetch=1,
        grid=(_NBLK, 4),
        in_specs=[
            pl.BlockSpec((_BLK, _H // 4),
                         lambda b, k, be: (jnp.minimum(b, be[24] - 1), k)),
            pl.BlockSpec((1, _H // 4, _D), lambda b, k, be: (be[b], k, 0)),
            pl.BlockSpec((1, 1, _D), lambda b, k, be: (be[b], 0, 0)),
            pl.BlockSpec((_BLK, 128),
                         lambda b, k, be: (jnp.minimum(b, be[24] - 1), 0)),
        ],
        out_specs=pl.BlockSpec(
            (_BLK, _D),
            lambda b, k, be: (jnp.where(b < be[24], b, _NBLK - 1), 0)),
        scratch_shapes=[pltpu.VMEM((_BLK, _D), jnp.float32)],
    ),
    out_shape=jax.ShapeDtypeStruct((_NSLOT, _D), jnp.float32),
    compiler_params=pltpu.CompilerParams(
        dimension_semantics=("arbitrary", "arbitrary")),
)


@functools.cache
def _make_combine():
    mesh = plsc.VectorSubcoreMesh(core_axis_name="c", subcore_axis_name="s")

    @functools.partial(
        pl.kernel,
        out_type=jax.ShapeDtypeStruct((_N, _D), jnp.float32),
        mesh=mesh,
        scratch_types=[
            pltpu.VMEM((16,), jnp.int32),
            pltpu.VMEM((16,), jnp.int32),
            pltpu.VMEM((16,), jnp.int32),
            pltpu.VMEM((16,), jnp.int32),
            pltpu.VMEM((16, _D), jnp.float32),
            pltpu.VMEM((16, _D), jnp.float32),
            pltpu.VMEM((16, _D), jnp.float32),
            pltpu.VMEM((16, _D), jnp.float32),
            pltpu.SemaphoreType.DMA,
            pltpu.SemaphoreType.DMA,
            pltpu.SemaphoreType.DMA,
            pltpu.SemaphoreType.DMA,
        ],
    )
    def _combine(ys_hbm, pp_hbm, out_hbm,
                 ia0, ia1, ib0, ib1, ra0, ra1, rb0, rb1,
                 sa0, sa1, sb0, sb1):
        cid = lax.axis_index("c")
        sid = lax.axis_index("s")
        wid = sid * 2 + cid
        idxs = ((ia0, ia1), (ib0, ib1))
        rows = ((ra0, ra1), (rb0, rb1))
        sems = ((sa0, sa1), (sb0, sb1))

        def gather(ch, st):
            tok0 = wid * 64 + ch * 16
            row = tok0 // 128
            col = tok0 % 128
            pltpu.sync_copy(pp_hbm.at[row, pl.ds(col, 16)], idxs[st][0])
            pltpu.sync_copy(pp_hbm.at[16 + row, pl.ds(col, 16)], idxs[st][1])
            return (
                pltpu.async_copy(ys_hbm.at[idxs[st][0]], rows[st][0],
                                 sems[st][0]),
                pltpu.async_copy(ys_hbm.at[idxs[st][1]], rows[st][1],
                                 sems[st][1]),
            )

        cps = {0: gather(0, 0)}
        for c in range(4):
            st = c % 2
            cps[c][0].wait()
            cps[c][1].wait()
            if c + 1 < 4:
                cps[c + 1] = gather(c + 1, 1 - st)
            r0, r1 = rows[st]

            def row_body(r, carry):
                def col_body(cc, inner):
                    sl = pl.ds(cc * 16, 16)
                    r0[r, sl] = r0[r, sl] + r1[r, sl]
                    return inner
                return lax.fori_loop(0, _D // 16, col_body, carry,
                                     unroll=8)

            lax.fori_loop(0, 16, row_body, 0)
            pltpu.sync_copy(r0, out_hbm.at[pl.ds(wid * 64 + c * 16, 16)])

    return _combine


def kernel(x, Wr, br, W1, b1, W2, b2, W3, b3):
    bsz, t, d = x.shape
    xf = x.reshape(t, d)

    pos, grep, be, aux = _router(xf, Wr, br.reshape(1, _E))
    xs, gs = _make_dispatch()(xf, pos.reshape(32, 128),
                              grep.reshape(32, 128, 128))
    meta = be.reshape(32)
    g = _ffn_a(meta, xs, W1, b1.reshape(_E, 1, _H), W2, b2.reshape(_E, 1, _H))
    ys = _ffn_b(meta, g, W3, b3.reshape(_E, 1, _D), gs)
    out = _make_combine()(ys, pos.reshape(32, 128))
    return out.reshape(bsz, t, d), aux[0, 0]


# speedup vs baseline: 1.2185x; 1.2185x over previous
import functools

import jax
import jax.numpy as jnp
from jax import lax
from jax.experimental import pallas as pl
from jax.experimental.pallas import tpu as pltpu
from jax.experimental.pallas import tpu_sc as plsc

_D = 1024
_H = 4096
_E = 8
_N = 2048
_NP = 2 * _N
_BLK = 256
_NBLK = 24
_NSLOT = _NBLK * _BLK
_NH = 4
_HT = _H // _NH


def _router_body(xf_ref, wr_ref, br_ref, pos_ref, grep_ref, be_ref, aux_ref):
    xf = xf_ref[...]
    logits = jnp.dot(xf, wr_ref[...], preferred_element_type=jnp.float32)
    logits = logits + br_ref[...]
    m = jnp.max(logits, axis=1, keepdims=True)
    ex = jnp.exp(logits - m)
    probs = ex / jnp.sum(ex, axis=1, keepdims=True)

    lane = lax.broadcasted_iota(jnp.int32, (_N, _E), 1)
    i1 = jnp.argmax(probs, axis=1, keepdims=True).astype(jnp.int32)
    v1 = jnp.max(probs, axis=1, keepdims=True)
    masked = jnp.where(lane == i1, -1.0, probs)
    i2 = jnp.argmax(masked, axis=1, keepdims=True).astype(jnp.int32)
    v2 = jnp.max(masked, axis=1, keepdims=True)
    tot = v1 + v2
    g0 = v1 / tot
    g1 = v2 / tot

    one1 = (lane == i1).astype(jnp.float32)
    f = jnp.sum(one1, axis=0, keepdims=True) * (1.0 / _N)
    pbar = jnp.sum(probs, axis=0, keepdims=True) * (1.0 / _N)
    aux_ref[...] = jnp.reshape(float(_E) * jnp.sum(f * pbar), (1, 1))

    e_pair = jnp.concatenate([i1, i2], axis=0)
    g_pair = jnp.concatenate([g0, g1], axis=0)
    lane_p = lax.broadcasted_iota(jnp.int32, (_NP, _E), 1)
    oh = (lane_p == e_pair).astype(jnp.float32)

    counts = jnp.sum(oh, axis=0, keepdims=True)
    padded = jnp.ceil(counts * (1.0 / _BLK)) * float(_BLK)
    ia = lax.broadcasted_iota(jnp.int32, (_E, _E), 0)
    ja = lax.broadcasted_iota(jnp.int32, (_E, _E), 1)
    tri_e = (ia < ja).astype(jnp.float32)
    base = jnp.dot(padded, tri_e, preferred_element_type=jnp.float32)

    ch = 512
    ic = lax.broadcasted_iota(jnp.int32, (ch, ch), 0)
    jc = lax.broadcasted_iota(jnp.int32, (ch, ch), 1)
    tri_c = (jc < ic).astype(jnp.float32)
    carry = jnp.zeros((1, _E), jnp.float32)
    ranks = []
    for c in range(_NP // ch):
        blk = lax.slice(oh, (c * ch, 0), ((c + 1) * ch, _E))
        ranks.append(jnp.dot(tri_c, blk, preferred_element_type=jnp.float32)
                     + carry)
        carry = carry + jnp.sum(blk, axis=0, keepdims=True)
    rank = jnp.concatenate(ranks, axis=0)

    posf = jnp.sum((rank + base) * oh, axis=1, keepdims=True)
    pos_ref[...] = posf.astype(jnp.int32)
    grep_ref[...] = g_pair * jnp.ones((1, 128), jnp.float32)

    bv = (lax.broadcasted_iota(jnp.int32, (32, _E), 0) * _BLK).astype(
        jnp.float32)
    be = jnp.sum((base <= bv).astype(jnp.int32), axis=1, keepdims=True) - 1
    lane_e = lax.broadcasted_iota(jnp.int32, (32, _E), 1)
    has = (jnp.ones((32, 1), jnp.float32) * counts) > 0.0
    elast = jnp.max(jnp.where(has, lane_e, 0), axis=1, keepdims=True)
    nreal = jnp.sum(padded, axis=1, keepdims=True) * (1.0 / _BLK)
    row = lax.broadcasted_iota(jnp.int32, (32, 1), 0)
    meta = jnp.where(row == 24, nreal.astype(jnp.int32)[0:1, 0:1],
                     jnp.minimum(be, elast))
    be_ref[...] = meta


_router = pl.pallas_call(
    _router_body,
    out_shape=(
        jax.ShapeDtypeStruct((_NP, 1), jnp.int32),
        jax.ShapeDtypeStruct((_NP, 128), jnp.float32),
        jax.ShapeDtypeStruct((32, 1), jnp.int32),
        jax.ShapeDtypeStruct((1, 1), jnp.float32),
    ),
)


@functools.cache
def _make_dispatch():
    mesh = plsc.VectorSubcoreMesh(core_axis_name="c", subcore_axis_name="s")

    @functools.partial(
        pl.kernel,
        out_type=(
            jax.ShapeDtypeStruct((_NSLOT, _D), jnp.float32),
            jax.ShapeDtypeStruct((_NSLOT, 128), jnp.float32),
        ),
        mesh=mesh,
        scratch_types=[
            pltpu.VMEM((32,), jnp.int32),
            pltpu.VMEM((32,), jnp.int32),
            pltpu.VMEM((32, _D), jnp.float32),
            pltpu.VMEM((32, _D), jnp.float32),
            pltpu.VMEM((128,), jnp.int32),
            pltpu.VMEM((128, 128), jnp.float32),
            pltpu.SemaphoreType.DMA,
            pltpu.SemaphoreType.DMA,
            pltpu.SemaphoreType.DMA,
        ],
    )
    def _dispatch(xf_hbm, pos_hbm, grep_hbm, xs_hbm, gs_hbm,
                  idx0_v, idx1_v, rows0_v, rows1_v, idxg_v, grows_v,
                  sem0, sem1, semg):
        cid = lax.axis_index("c")
        sid = lax.axis_index("s")
        wid = sid * 2 + cid
        src0 = (wid % 16) * 128

        pltpu.sync_copy(pos_hbm.at[wid], idxg_v)
        pltpu.sync_copy(grep_hbm.at[wid], grows_v)
        gcp = pltpu.async_copy(grows_v, gs_hbm.at[idxg_v], semg)

        idxs = (idx0_v, idx1_v)
        rows = (rows0_v, rows1_v)
        sems = (sem0, sem1)
        cps = [None, None]
        for c in range(4):
            bi = c % 2
            if cps[bi] is not None:
                cps[bi].wait()
            pltpu.sync_copy(pos_hbm.at[wid, pl.ds(c * 32, 32)], idxs[bi])
            pltpu.sync_copy(xf_hbm.at[pl.ds(src0 + c * 32, 32)], rows[bi])
            cps[bi] = pltpu.async_copy(rows[bi], xs_hbm.at[idxs[bi]], sems[bi])
        gcp.wait()
        cps[0].wait()
        cps[1].wait()

    return _dispatch


_NHA = 2
_HTA = _H // _NHA


def _ffn_a_body(be_ref, xs_ref, w1_ref, b1_ref, w2_ref, b2_ref, g_ref):
    b = pl.program_id(1)

    @pl.when(b < be_ref[24])
    def _():
        xv = xs_ref[...]
        h1 = jnp.dot(xv, w1_ref[0], preferred_element_type=jnp.float32)
        h1 = h1 + b1_ref[0]
        h2 = jnp.dot(xv, w2_ref[0], preferred_element_type=jnp.float32)
        h2 = h2 + b2_ref[0]
        g_ref[...] = (h1 * jax.nn.sigmoid(h1) * h2).astype(jnp.bfloat16)


_ffn_a = pl.pallas_call(
    _ffn_a_body,
    grid_spec=pltpu.PrefetchScalarGridSpec(
        num_scalar_prefetch=1,
        grid=(_NHA, _NBLK),
        in_specs=[
            pl.BlockSpec((_BLK, _D),
                         lambda h, b, be: (jnp.minimum(b, be[24] - 1), 0)),
            pl.BlockSpec((1, _D, _HTA), lambda h, b, be: (be[b], 0, h)),
            pl.BlockSpec((1, 1, _HTA), lambda h, b, be: (be[b], 0, h)),
            pl.BlockSpec((1, _D, _HTA), lambda h, b, be: (be[b], 0, h)),
            pl.BlockSpec((1, 1, _HTA), lambda h, b, be: (be[b], 0, h)),
        ],
        out_specs=pl.BlockSpec(
            (_BLK, _HTA),
            lambda h, b, be: (jnp.where(b < be[24], b, _NBLK - 1), h)),
    ),
    out_shape=jax.ShapeDtypeStruct((_NSLOT, _H), jnp.bfloat16),
    compiler_params=pltpu.CompilerParams(
        dimension_semantics=("arbitrary", "arbitrary")),
)


def _ffn_b_body(be_ref, g_ref, w3_ref, b3_ref, gs_ref, ys_ref):
    b = pl.program_id(0)

    @pl.when(b < be_ref[24])
    def _():
        y = jnp.dot(g_ref[...], w3_ref[0], preferred_element_type=jnp.float32)
        ys_ref[...] = (y + b3_ref[0]) * gs_ref[:, 0:1]


_ffn_b = pl.pallas_call(
    _ffn_b_body,
    grid_spec=pltpu.PrefetchScalarGridSpec(
        num_scalar_prefetch=1,
        grid=(_NBLK,),
        in_specs=[
            pl.BlockSpec((_BLK, _H),
                         lambda b, be: (jnp.minimum(b, be[24] - 1), 0)),
            pl.BlockSpec((1, _H, _D), lambda b, be: (be[b], 0, 0)),
            pl.BlockSpec((1, 1, _D), lambda b, be: (be[b], 0, 0)),
            pl.BlockSpec((_BLK, 128),
                         lambda b, be: (jnp.minimum(b, be[24] - 1), 0)),
        ],
        out_specs=pl.BlockSpec(
            (_BLK, _D),
            lambda b, be: (jnp.where(b < be[24], b, _NBLK - 1), 0)),
    ),
    out_shape=jax.ShapeDtypeStruct((_NSLOT, _D), jnp.float32),
    compiler_params=pltpu.CompilerParams(
        dimension_semantics=("arbitrary",)),
)


@functools.cache
def _make_combine():
    mesh = plsc.VectorSubcoreMesh(core_axis_name="c", subcore_axis_name="s")

    @functools.partial(
        pl.kernel,
        out_type=jax.ShapeDtypeStruct((_N, _D), jnp.float32),
        mesh=mesh,
        scratch_types=[
            pltpu.VMEM((16,), jnp.int32),
            pltpu.VMEM((16,), jnp.int32),
            pltpu.VMEM((16,), jnp.int32),
            pltpu.VMEM((16,), jnp.int32),
            pltpu.VMEM((16, _D), jnp.float32),
            pltpu.VMEM((16, _D), jnp.float32),
            pltpu.VMEM((16, _D), jnp.float32),
            pltpu.VMEM((16, _D), jnp.float32),
            pltpu.SemaphoreType.DMA,
            pltpu.SemaphoreType.DMA,
            pltpu.SemaphoreType.DMA,
            pltpu.SemaphoreType.DMA,
        ],
    )
    def _combine(ys_hbm, pp_hbm, out_hbm,
                 ia0, ia1, ib0, ib1, ra0, ra1, rb0, rb1,
                 sa0, sa1, sb0, sb1):
        cid = lax.axis_index("c")
        sid = lax.axis_index("s")
        wid = sid * 2 + cid
        idxs = ((ia0, ia1), (ib0, ib1))
        rows = ((ra0, ra1), (rb0, rb1))
        sems = ((sa0, sa1), (sb0, sb1))

        def gather(ch, st):
            tok0 = wid * 64 + ch * 16
            row = tok0 // 128
            col = tok0 % 128
            pltpu.sync_copy(pp_hbm.at[row, pl.ds(col, 16)], idxs[st][0])
            pltpu.sync_copy(pp_hbm.at[16 + row, pl.ds(col, 16)], idxs[st][1])
            return (
                pltpu.async_copy(ys_hbm.at[idxs[st][0]], rows[st][0],
                                 sems[st][0]),
                pltpu.async_copy(ys_hbm.at[idxs[st][1]], rows[st][1],
                                 sems[st][1]),
            )

        cps = {0: gather(0, 0)}
        for c in range(4):
            st = c % 2
            cps[c][0].wait()
            cps[c][1].wait()
            if c + 1 < 4:
                cps[c + 1] = gather(c + 1, 1 - st)
            r0, r1 = rows[st]

            def row_body(r, carry):
                def col_body(cc, inner):
                    sl = pl.ds(cc * 16, 16)
                    r0[r, sl] = r0[r, sl] + r1[r, sl]
                    return inner
                return lax.fori_loop(0, _D // 16, col_body, carry,
                                     unroll=8)

            lax.fori_loop(0, 16, row_body, 0)
            pltpu.sync_copy(r0, out_hbm.at[pl.ds(wid * 64 + c * 16, 16)])

    return _combine


def kernel(x, Wr, br, W1, b1, W2, b2, W3, b3):
    bsz, t, d = x.shape
    xf = x.reshape(t, d)

    pos, grep, be, aux = _router(xf, Wr, br.reshape(1, _E))
    xs, gs = _make_dispatch()(xf, pos.reshape(32, 128),
                              grep.reshape(32, 128, 128))
    meta = be.reshape(32)
    g = _ffn_a(meta, xs, W1, b1.reshape(_E, 1, _H), W2, b2.reshape(_E, 1, _H))
    ys = _ffn_b(meta, g, W3, b3.reshape(_E, 1, _D), gs)
    out = _make_combine()(ys, pos.reshape(32, 128))
    return out.reshape(bsz, t, d), aux[0, 0]
